# baseline (speedup 1.0000x reference)
import functools

import jax
import jax.numpy as jnp
from jax import lax
from jax.experimental import pallas as pl
from jax.experimental.pallas import tpu as pltpu
from jax.experimental.pallas import tpu_sc as plsc

N = 10000
E = 320000
IN_DIM = 128
HID = 64

NC = 2
NS = 16
NPAD = 10240
RPT = NPAD // NS
EB = 112
EPAD = 322560
F0 = 200704
W = 128

RB = 512
NRB = (N + RB - 1) // RB

f32 = jnp.float32


def _make_sc_agg(col_split):
  mesh = plsc.VectorSubcoreMesh(core_axis_name="c", subcore_axis_name="s")
  if col_split:
    eps = EPAD // NS
    nblk = eps // EB
  else:
    eps0 = F0 // NS
    eps1 = (EPAD - F0) // NS
    nblk0 = eps0 // EB
    nblk1 = eps1 // EB

  @functools.partial(
      pl.kernel,
      out_type=jax.ShapeDtypeStruct((NC, NPAD, W), f32),
      mesh=mesh,
      scratch_types=[
          pltpu.VMEM((EB,), jnp.int32),
          pltpu.VMEM((EB,), jnp.int32),
          pltpu.VMEM((EB,), jnp.int32),
          pltpu.VMEM((EB,), jnp.int32),
          pltpu.VMEM((EB, W), f32),
          pltpu.VMEM((EB, W), f32),
          pltpu.VMEM_SHARED((NPAD, W), f32),
          pltpu.SemaphoreType.DMA,
          pltpu.SemaphoreType.DMA,
      ],
  )
  def agg(hp_hbm, src_hbm, dst_hbm, zeros_hbm, out_hbm,
          src0_v, src1_v, dst0_v, dst1_v, buf0_v, buf1_v, acc,
          gsem0, gsem1):
    c = lax.axis_index("c")
    s = lax.axis_index("s")
    pltpu.sync_copy(zeros_hbm, acc.at[pl.ds(s * RPT, RPT)])
    plsc.subcore_barrier()

    def offs(k):
      if col_split:
        base = s * eps + k * EB
        src_off = c * EPAD + base
      else:
        base = jnp.where(c == 0, s * eps0 + k * EB,
                         F0 + s * eps1 + k * EB)
        src_off = base
      return src_off, base

    def load_fire(k, src_v, dst_v, buf_v, sem):
      src_off, base = offs(k)
      pltpu.sync_copy(src_hbm.at[pl.ds(src_off, EB)], src_v)
      pltpu.sync_copy(dst_hbm.at[pl.ds(base, EB)], dst_v)
      pltpu.async_copy(hp_hbm.at[src_v], buf_v, sem)

    load_fire(0, src0_v, dst0_v, buf0_v, gsem0)

    def body(g, carry):
      load_fire(2 * g + 1, src1_v, dst1_v, buf1_v, gsem1)
      pltpu.make_async_copy(hp_hbm.at[pl.ds(0, EB)], buf0_v, gsem0).wait()
      pltpu.sync_copy(buf0_v, acc.at[dst0_v], add=True)
      load_fire(2 * g + 2, src0_v, dst0_v, buf0_v, gsem0)
      pltpu.make_async_copy(hp_hbm.at[pl.ds(0, EB)], buf1_v, gsem1).wait()
      pltpu.sync_copy(buf1_v, acc.at[dst1_v], add=True)
      return carry

    ngrp = (nblk // 2) if col_split else jnp.where(c == 0, nblk0 // 2,
                                                  nblk1 // 2)
    lax.fori_loop(0, ngrp, body, 0)
    pltpu.make_async_copy(hp_hbm.at[pl.ds(0, EB)], buf0_v, gsem0).wait()
    plsc.subcore_barrier()
    pltpu.sync_copy(acc.at[pl.ds(s * RPT, RPT)],
                    out_hbm.at[c, pl.ds(s * RPT, RPT)])

  return agg


def _make_sc_deg():
  mesh = plsc.VectorSubcoreMesh(core_axis_name="c", subcore_axis_name="s")
  eps = EPAD // (NC * NS)
  nblk = eps // EB

  @functools.partial(
      pl.kernel,
      out_type=jax.ShapeDtypeStruct((NC, NPAD, W), f32),
      mesh=mesh,
      scratch_types=[
          pltpu.VMEM((EB,), jnp.int32),
          pltpu.VMEM((EB, W), f32),
          pltpu.VMEM_SHARED((NPAD, W), f32),
      ],
  )
  def deg(ones_hbm, dst_hbm, zeros_hbm, out_hbm, dst_v, ones_v, acc):
    c = lax.axis_index("c")
    s = lax.axis_index("s")
    pltpu.sync_copy(zeros_hbm, acc.at[pl.ds(s * RPT, RPT)])
    pltpu.sync_copy(ones_hbm, ones_v)
    plsc.subcore_barrier()

    def body(k, carry):
      base = (c * NS + s) * eps + k * EB
      pltpu.sync_copy(dst_hbm.at[pl.ds(base, EB)], dst_v)
      pltpu.sync_copy(ones_v, acc.at[dst_v], add=True)
      return carry

    lax.fori_loop(0, nblk, body, 0)
    plsc.subcore_barrier()
    pltpu.sync_copy(acc.at[pl.ds(s * RPT, RPT)],
                    out_hbm.at[c, pl.ds(s * RPT, RPT)])

  return deg


def _full(shape):
  return pl.BlockSpec(shape, lambda i: tuple(0 for _ in shape))


def _rows(shape):
  rank = len(shape)
  return pl.BlockSpec(shape, lambda i: (i,) + tuple(0 for _ in range(rank - 1)))


def _halves(shape):
  return pl.BlockSpec(shape, lambda i: (0, i, 0))


def _cat2(ref):
  return jnp.concatenate([ref[0], ref[1]], axis=1)


def _sum2(ref):
  return ref[0] + ref[1]


def _tc_a_body(dp_ref, x_ref, we1_ref, dinv_ref, hp1_ref):
  deg = dp_ref[0, :, 0] + dp_ref[1, :, 0] + 1.0
  dinv = lax.rsqrt(deg)[:, None]
  x = x_ref[...]
  h = jnp.dot(x, we1_ref[...], preferred_element_type=f32)
  q0 = dinv * (1.0 - 2.0 * x[:, 0:1])
  wide = jnp.concatenate(
      [h * dinv, q0, jnp.zeros((RB, W - HID - 1), f32)], axis=1)
  dinv_ref[...] = dinv
  hp1_ref[...] = wide


def _tc_b_body(agg1_ref, hp1_ref, dinv_ref, we1_ref, w2_ref, be1_ref,
               hp2_ref):
  a = _sum2(agg1_ref) + hp1_ref[...]
  dinv = dinv_ref[...]
  pre1 = dinv * a[:, :64] + be1_ref[...][None, :]
  q = dinv * a[:, 64:65]
  h1 = jnp.maximum(pre1, 0.0)
  h1cf = jnp.maximum(pre1 + q * we1_ref[0:1, :], 0.0)
  w2 = w2_ref[...]
  h2 = jnp.concatenate(
      [jnp.dot(h1, w2, preferred_element_type=f32),
       jnp.dot(h1cf, w2[:, :32], preferred_element_type=f32),
       jnp.zeros((RB, 32), f32)], axis=1)
  hp2_ref[...] = dinv * h2


def _tc_c_body(agg2_ref, hp2_ref, dinv_ref, be2_ref, zs_ref, zns_ref,
               hp3_ref):
  a = _sum2(agg2_ref) + hp2_ref[...]
  dinv = dinv_ref[...]
  s2 = dinv * a
  be2 = be2_ref[...]
  z = s2[:, :64] + be2[None, :]
  z_s = z[:, :32]
  z_ns = z[:, 32:]
  z_s_cf = s2[:, 64:96] + be2[None, :32]
  zs_ref[...] = z_s
  zns_ref[...] = z_ns
  hp3_ref[...] = dinv * jnp.concatenate(
      [z_s, z_ns, z_s_cf, jnp.zeros((RB, 32), f32)], axis=1)


def _tc_d_body(agg3_ref, hp3_ref, dinv_ref, wa11_ref, wa21_ref, ws_ref,
               ba11_ref, ba21_ref, bs_ref, hs_ref, hp4a_ref, hp4b_ref):
  v = dinv_ref[...] * (_sum2(agg3_ref) + hp3_ref[...])
  p_zs = v[:, 0:32]
  p_zns = v[:, 32:64]
  p_zscf = v[:, 64:96]
  wa11 = wa11_ref[...]
  ba11 = ba11_ref[...][None, :]
  u1 = jnp.maximum(jnp.dot(p_zs, wa11, preferred_element_type=f32) + ba11,
                   0.0)
  u2 = jnp.maximum(jnp.dot(p_zns, wa21_ref[...], preferred_element_type=f32)
                   + ba21_ref[...][None, :], 0.0)
  u3 = jnp.maximum(jnp.dot(p_zscf, wa11, preferred_element_type=f32) + ba11,
                   0.0)
  hs_ref[...] = (jnp.dot(p_zns, ws_ref[...], preferred_element_type=f32)
                 + bs_ref[...][None, :])
  dinv = dinv_ref[...]
  hp4a_ref[...] = dinv * jnp.concatenate([u1, u2], axis=1)
  hp4b_ref[...] = dinv * jnp.concatenate([u3, jnp.zeros((RB, 64), f32)],
                                         axis=1)


def _tc_e_body(agg4a_ref, hp4a_ref, agg4b_ref, hp4b_ref, dinv_ref,
               wa12_ref, wa22_ref, ba12_ref, ba22_ref,
               xs_ref, xns_ref, xscf_ref):
  dinv = dinv_ref[...]
  va = dinv * (_sum2(agg4a_ref) + hp4a_ref[...])
  vb = dinv * (_sum2(agg4b_ref) + hp4b_ref[...])
  wa12 = wa12_ref[...]
  ba12 = ba12_ref[...][None, :]
  xs_ref[...] = (jnp.dot(va[:, 0:64], wa12, preferred_element_type=f32)
                 + ba12)
  xns_ref[...] = (jnp.dot(va[:, 64:128], wa22_ref[...],
                          preferred_element_type=f32)
                  + ba22_ref[...][None, :])
  xscf_ref[...] = (jnp.dot(vb[:, 0:64], wa12, preferred_element_type=f32)
                   + ba12)


def _tc_f_body(a_ref, b_ref, out_ref):
  out_ref[...] = lax.dot_general(
      a_ref[...], b_ref[...], (((1,), (1,)), ((), ())),
      preferred_element_type=f32)


def kernel(x, W_e1, b_e1, W_e2, b_e2, Wa11, ba11, Wa12, ba12, Wa21, ba21,
           Wa22, ba22, Ws, bs, edge_index):
  pad = jnp.full((EPAD - E,), N, jnp.int32)
  padd = N + 8 + (jnp.arange(EPAD - E, dtype=jnp.int32) % (NPAD - N - 8))
  ext = jnp.full((EB,), N, jnp.int32)
  src = jnp.concatenate([edge_index[0].astype(jnp.int32), pad, ext])
  dst = jnp.concatenate([edge_index[1].astype(jnp.int32), padd, ext + 8])
  src2 = jnp.concatenate([src[:EPAD], src[:EPAD] + NPAD, ext])
  zrows = jnp.zeros((RPT, W), f32)

  agg_es = _make_sc_agg(col_split=False)
  agg_cs = _make_sc_agg(col_split=True)

  degpart = _make_sc_deg()(jnp.ones((EB, W), f32), dst, zrows)

  tc_a = pl.pallas_call(
      _tc_a_body,
      grid=(NRB,),
      in_specs=[_halves((2, RB, W)),
                _rows((RB, IN_DIM)), _full((IN_DIM, HID))],
      out_specs=[_rows((RB, 1)), _rows((RB, W))],
      out_shape=[jax.ShapeDtypeStruct((N, 1), f32),
                 jax.ShapeDtypeStruct((NPAD, W), f32)],
  )
  dinv, hp1 = tc_a(degpart, x, W_e1)

  agg1 = agg_es(hp1, src, dst, zrows)

  tc_b = pl.pallas_call(
      _tc_b_body,
      grid=(NRB,),
      in_specs=[_halves((2, RB, W)), _rows((RB, W)), _rows((RB, 1)),
                _full((IN_DIM, HID)), _full((HID, HID)), _full((HID,))],
      out_specs=[_rows((RB, W))],
      out_shape=[jax.ShapeDtypeStruct((NPAD, W), f32)],
  )
  (hp2,) = tc_b(agg1, hp1, dinv, W_e1, W_e2, b_e1)

  agg2 = agg_es(hp2, src, dst, zrows)

  h = HID // 2
  tc_c = pl.pallas_call(
      _tc_c_body,
      grid=(NRB,),
      in_specs=[_halves((2, RB, W)), _rows((RB, W)), _rows((RB, 1)),
                _full((HID,))],
      out_specs=[_rows((RB, h)), _rows((RB, h)), _rows((RB, W))],
      out_shape=[jax.ShapeDtypeStruct((N, h), f32),
                 jax.ShapeDtypeStruct((N, h), f32),
                 jax.ShapeDtypeStruct((NPAD, W), f32)],
  )
  z_s, z_ns, hp3 = tc_c(agg2, hp2, dinv, b_e2)

  agg3 = agg_es(hp3, src, dst, zrows)

  tc_d = pl.pallas_call(
      _tc_d_body,
      grid=(NRB,),
      in_specs=[_halves((2, RB, W)), _rows((RB, W)), _rows((RB, 1)),
                _full((h, HID)), _full((h, HID)), _full((h, HID)),
                _full((HID,)), _full((HID,)), _full((HID,))],
      out_specs=[_rows((RB, HID)), _rows((RB, W)), _rows((RB, W))],
      out_shape=[jax.ShapeDtypeStruct((N, HID), f32),
                 jax.ShapeDtypeStruct((NPAD, W), f32),
                 jax.ShapeDtypeStruct((NPAD, W), f32)],
  )
  hs, hp4a, hp4b = tc_d(agg3, hp3, dinv, Wa11, Wa21, Ws, ba11, ba21, bs)

  CB = 1280
  tc_f = pl.pallas_call(
      _tc_f_body,
      grid=(NRB, pl.cdiv(N, CB)),
      in_specs=[pl.BlockSpec((RB, HID), lambda i, j: (i, 0)),
                pl.BlockSpec((CB, HID), lambda i, j: (j, 0))],
      out_specs=pl.BlockSpec((RB, CB), lambda i, j: (i, j)),
      out_shape=jax.ShapeDtypeStruct((N, N), f32),
  )
  s_ = tc_f(hs, hs)

  agg4a = agg_es(hp4a, src, dst, zrows)
  agg4b = agg_es(hp4b, src, dst, zrows)

  tc_e = pl.pallas_call(
      _tc_e_body,
      grid=(NRB,),
      in_specs=[_halves((2, RB, W)), _rows((RB, W)),
                _halves((2, RB, W)), _rows((RB, W)), _rows((RB, 1)),
                _full((HID, IN_DIM)), _full((HID, IN_DIM)),
                _full((IN_DIM,)), _full((IN_DIM,))],
      out_specs=[_rows((RB, IN_DIM))] * 3,
      out_shape=[jax.ShapeDtypeStruct((N, IN_DIM), f32)] * 3,
  )
  x_s_hat, x_ns_hat, x_s_cf_hat = tc_e(agg4a, hp4a, agg4b, hp4b, dinv,
                                       Wa12, Wa22, ba12, ba22)

  return (z_s, z_ns, x_s_hat, x_ns_hat, x_s_cf_hat, s_)


# speedup vs baseline: 1.0764x; 1.0764x over previous
import functools

import jax
import jax.numpy as jnp
from jax import lax
from jax.experimental import pallas as pl
from jax.experimental.pallas import tpu as pltpu
from jax.experimental.pallas import tpu_sc as plsc

N = 10000
E = 320000
IN_DIM = 128
HID = 64

NC = 2
NS = 16
NPAD = 10240
RPT = NPAD // NS
EB = 112
EPAD = 322560
F0 = 200704
W = 128

RB = 512
NRB = (N + RB - 1) // RB

f32 = jnp.float32


def _make_sc_agg(col_split):
  mesh = plsc.VectorSubcoreMesh(core_axis_name="c", subcore_axis_name="s")
  if col_split:
    eps = EPAD // NS
    nblk = eps // EB
  else:
    eps0 = F0 // NS
    eps1 = (EPAD - F0) // NS
    nblk0 = eps0 // EB
    nblk1 = eps1 // EB

  @functools.partial(
      pl.kernel,
      out_type=jax.ShapeDtypeStruct((NC, NPAD, W), f32),
      mesh=mesh,
      scratch_types=[
          pltpu.VMEM((EB,), jnp.int32),
          pltpu.VMEM((EB,), jnp.int32),
          pltpu.VMEM((EB,), jnp.int32),
          pltpu.VMEM((EB,), jnp.int32),
          pltpu.VMEM((EB, W), f32),
          pltpu.VMEM((EB, W), f32),
          pltpu.VMEM_SHARED((NPAD, W), f32),
          pltpu.SemaphoreType.DMA,
          pltpu.SemaphoreType.DMA,
      ],
  )
  def agg(hp_hbm, src_hbm, dst_hbm, zeros_hbm, out_hbm,
          src0_v, src1_v, dst0_v, dst1_v, buf0_v, buf1_v, acc,
          gsem0, gsem1):
    c = lax.axis_index("c")
    s = lax.axis_index("s")
    pltpu.sync_copy(zeros_hbm, acc.at[pl.ds(s * RPT, RPT)])
    plsc.subcore_barrier()

    def offs(k):
      if col_split:
        base = s * eps + k * EB
        src_off = c * EPAD + base
      else:
        base = jnp.where(c == 0, s * eps0 + k * EB,
                         F0 + s * eps1 + k * EB)
        src_off = base
      return src_off, base

    def load_fire(k, src_v, dst_v, buf_v, sem):
      src_off, base = offs(k)
      pltpu.sync_copy(src_hbm.at[pl.ds(src_off, EB)], src_v)
      pltpu.sync_copy(dst_hbm.at[pl.ds(base, EB)], dst_v)
      pltpu.async_copy(hp_hbm.at[src_v], buf_v, sem)

    load_fire(0, src0_v, dst0_v, buf0_v, gsem0)

    def body(g, carry):
      load_fire(2 * g + 1, src1_v, dst1_v, buf1_v, gsem1)
      pltpu.make_async_copy(hp_hbm.at[pl.ds(0, EB)], buf0_v, gsem0).wait()
      pltpu.sync_copy(buf0_v, acc.at[dst0_v], add=True)
      load_fire(2 * g + 2, src0_v, dst0_v, buf0_v, gsem0)
      pltpu.make_async_copy(hp_hbm.at[pl.ds(0, EB)], buf1_v, gsem1).wait()
      pltpu.sync_copy(buf1_v, acc.at[dst1_v], add=True)
      return carry

    ngrp = (nblk // 2) if col_split else jnp.where(c == 0, nblk0 // 2,
                                                  nblk1 // 2)
    lax.fori_loop(0, ngrp, body, 0)
    pltpu.make_async_copy(hp_hbm.at[pl.ds(0, EB)], buf0_v, gsem0).wait()
    plsc.subcore_barrier()
    pltpu.sync_copy(acc.at[pl.ds(s * RPT, RPT)],
                    out_hbm.at[c, pl.ds(s * RPT, RPT)])

  return agg


def _make_sc_deg():
  mesh = plsc.VectorSubcoreMesh(core_axis_name="c", subcore_axis_name="s")
  eps = EPAD // (NC * NS)
  nblk = eps // EB

  @functools.partial(
      pl.kernel,
      out_type=jax.ShapeDtypeStruct((NC, NPAD, W), f32),
      mesh=mesh,
      scratch_types=[
          pltpu.VMEM((EB,), jnp.int32),
          pltpu.VMEM((EB, W), f32),
          pltpu.VMEM_SHARED((NPAD, W), f32),
      ],
  )
  def deg(ones_hbm, dst_hbm, zeros_hbm, out_hbm, dst_v, ones_v, acc):
    c = lax.axis_index("c")
    s = lax.axis_index("s")
    pltpu.sync_copy(zeros_hbm, acc.at[pl.ds(s * RPT, RPT)])
    pltpu.sync_copy(ones_hbm, ones_v)
    plsc.subcore_barrier()

    def body(k, carry):
      base = (c * NS + s) * eps + k * EB
      pltpu.sync_copy(dst_hbm.at[pl.ds(base, EB)], dst_v)
      pltpu.sync_copy(ones_v, acc.at[dst_v], add=True)
      return carry

    lax.fori_loop(0, nblk, body, 0)
    plsc.subcore_barrier()
    pltpu.sync_copy(acc.at[pl.ds(s * RPT, RPT)],
                    out_hbm.at[c, pl.ds(s * RPT, RPT)])

  return deg


def _full(shape):
  return pl.BlockSpec(shape, lambda i: tuple(0 for _ in shape))


def _rows(shape):
  rank = len(shape)
  return pl.BlockSpec(shape, lambda i: (i,) + tuple(0 for _ in range(rank - 1)))


def _halves(shape):
  return pl.BlockSpec(shape, lambda i: (0, i, 0))


def _cat2(ref):
  return jnp.concatenate([ref[0], ref[1]], axis=1)


def _sum2(ref):
  return ref[0] + ref[1]


def _tc_a_body(dp_ref, x_ref, we1_ref, dinv_ref, hp1_ref):
  deg = dp_ref[0, :, 0] + dp_ref[1, :, 0] + 1.0
  dinv = lax.rsqrt(deg)[:, None]
  x = x_ref[...]
  h = jnp.dot(x, we1_ref[...], preferred_element_type=f32)
  q0 = dinv * (1.0 - 2.0 * x[:, 0:1])
  wide = jnp.concatenate(
      [h * dinv, q0, jnp.zeros((RB, W - HID - 1), f32)], axis=1)
  dinv_ref[...] = dinv
  hp1_ref[...] = wide


def _tc_b_body(agg1_ref, hp1_ref, dinv_ref, we1_ref, w2_ref, be1_ref,
               hp2_ref):
  a = _sum2(agg1_ref) + hp1_ref[...]
  dinv = dinv_ref[...]
  pre1 = dinv * a[:, :64] + be1_ref[...][None, :]
  q = dinv * a[:, 64:65]
  h1 = jnp.maximum(pre1, 0.0)
  h1cf = jnp.maximum(pre1 + q * we1_ref[0:1, :], 0.0)
  w2 = w2_ref[...]
  h2 = jnp.concatenate(
      [jnp.dot(h1, w2, preferred_element_type=f32),
       jnp.dot(h1cf, w2[:, :32], preferred_element_type=f32),
       jnp.zeros((RB, 32), f32)], axis=1)
  hp2_ref[...] = dinv * h2


def _tc_c_body(agg2_ref, hp2_ref, dinv_ref, be2_ref, zs_ref, zns_ref,
               hp3_ref):
  a = _sum2(agg2_ref) + hp2_ref[...]
  dinv = dinv_ref[...]
  s2 = dinv * a
  be2 = be2_ref[...]
  z = s2[:, :64] + be2[None, :]
  z_s = z[:, :32]
  z_ns = z[:, 32:]
  z_s_cf = s2[:, 64:96] + be2[None, :32]
  zs_ref[...] = z_s
  zns_ref[...] = z_ns
  hp3_ref[...] = dinv * jnp.concatenate(
      [z_s, z_ns, z_s_cf, jnp.zeros((RB, 32), f32)], axis=1)


def _tc_d_body(agg3_ref, hp3_ref, dinv_ref, wa11_ref, wa21_ref, ws_ref,
               ba11_ref, ba21_ref, bs_ref, hs_ref, hp4_ref):
  v = dinv_ref[...] * (_sum2(agg3_ref) + hp3_ref[...])
  p_zs = v[:, 0:32]
  p_zns = v[:, 32:64]
  p_zscf = v[:, 64:96]
  wa11 = wa11_ref[...]
  ba11 = ba11_ref[...][None, :]
  u1 = jnp.maximum(jnp.dot(p_zs, wa11, preferred_element_type=f32) + ba11,
                   0.0)
  u2 = jnp.maximum(jnp.dot(p_zns, wa21_ref[...], preferred_element_type=f32)
                   + ba21_ref[...][None, :], 0.0)
  u3 = jnp.maximum(jnp.dot(p_zscf, wa11, preferred_element_type=f32) + ba11,
                   0.0)
  hs_ref[...] = (jnp.dot(p_zns, ws_ref[...], preferred_element_type=f32)
                 + bs_ref[...][None, :])
  dinv = dinv_ref[...]
  hp4_ref[0] = dinv * jnp.concatenate([u1, u2], axis=1)
  hp4_ref[1] = dinv * jnp.concatenate([u3, jnp.zeros((RB, 64), f32)], axis=1)


def _tc_e_body(agg4_ref, hp4_ref, dinv_ref, wa12_ref, wa22_ref,
               ba12_ref, ba22_ref, xs_ref, xns_ref, xscf_ref):
  va = dinv_ref[...] * (_cat2(agg4_ref) + _cat2(hp4_ref))
  wa12 = wa12_ref[...]
  ba12 = ba12_ref[...][None, :]
  xs_ref[...] = (jnp.dot(va[:, 0:64], wa12, preferred_element_type=f32)
                 + ba12)
  xns_ref[...] = (jnp.dot(va[:, 64:128], wa22_ref[...],
                          preferred_element_type=f32)
                  + ba22_ref[...][None, :])
  xscf_ref[...] = (jnp.dot(va[:, 128:192], wa12, preferred_element_type=f32)
                   + ba12)


def _tc_f_body(a_ref, b_ref, out_ref):
  out_ref[...] = lax.dot_general(
      a_ref[...], b_ref[...], (((1,), (1,)), ((), ())),
      preferred_element_type=f32)


def kernel(x, W_e1, b_e1, W_e2, b_e2, Wa11, ba11, Wa12, ba12, Wa21, ba21,
           Wa22, ba22, Ws, bs, edge_index):
  pad = jnp.full((EPAD - E,), N, jnp.int32)
  padd = N + 8 + (jnp.arange(EPAD - E, dtype=jnp.int32) % (NPAD - N - 8))
  ext = jnp.full((EB,), N, jnp.int32)
  src = jnp.concatenate([edge_index[0].astype(jnp.int32), pad, ext])
  dst = jnp.concatenate([edge_index[1].astype(jnp.int32), padd, ext + 8])
  src2 = jnp.concatenate([src[:EPAD], src[:EPAD] + NPAD, ext])
  zrows = jnp.zeros((RPT, W), f32)

  agg_es = _make_sc_agg(col_split=False)
  agg_cs = _make_sc_agg(col_split=True)

  degpart = _make_sc_deg()(jnp.ones((EB, W), f32), dst, zrows)

  tc_a = pl.pallas_call(
      _tc_a_body,
      grid=(NRB,),
      in_specs=[_halves((2, RB, W)),
                _rows((RB, IN_DIM)), _full((IN_DIM, HID))],
      out_specs=[_rows((RB, 1)), _rows((RB, W))],
      out_shape=[jax.ShapeDtypeStruct((N, 1), f32),
                 jax.ShapeDtypeStruct((NPAD, W), f32)],
  )
  dinv, hp1 = tc_a(degpart, x, W_e1)

  agg1 = agg_es(hp1, src, dst, zrows)

  tc_b = pl.pallas_call(
      _tc_b_body,
      grid=(NRB,),
      in_specs=[_halves((2, RB, W)), _rows((RB, W)), _rows((RB, 1)),
                _full((IN_DIM, HID)), _full((HID, HID)), _full((HID,))],
      out_specs=[_rows((RB, W))],
      out_shape=[jax.ShapeDtypeStruct((NPAD, W), f32)],
  )
  (hp2,) = tc_b(agg1, hp1, dinv, W_e1, W_e2, b_e1)

  agg2 = agg_es(hp2, src, dst, zrows)

  h = HID // 2
  tc_c = pl.pallas_call(
      _tc_c_body,
      grid=(NRB,),
      in_specs=[_halves((2, RB, W)), _rows((RB, W)), _rows((RB, 1)),
                _full((HID,))],
      out_specs=[_rows((RB, h)), _rows((RB, h)), _rows((RB, W))],
      out_shape=[jax.ShapeDtypeStruct((N, h), f32),
                 jax.ShapeDtypeStruct((N, h), f32),
                 jax.ShapeDtypeStruct((NPAD, W), f32)],
  )
  z_s, z_ns, hp3 = tc_c(agg2, hp2, dinv, b_e2)

  agg3 = agg_es(hp3, src, dst, zrows)

  tc_d = pl.pallas_call(
      _tc_d_body,
      grid=(NRB,),
      in_specs=[_halves((2, RB, W)), _rows((RB, W)), _rows((RB, 1)),
                _full((h, HID)), _full((h, HID)), _full((h, HID)),
                _full((HID,)), _full((HID,)), _full((HID,))],
      out_specs=[_rows((RB, HID)), _halves((2, RB, W))],
      out_shape=[jax.ShapeDtypeStruct((N, HID), f32),
                 jax.ShapeDtypeStruct((2, NPAD, W), f32)],
  )
  hs, hp4 = tc_d(agg3, hp3, dinv, Wa11, Wa21, Ws, ba11, ba21, bs)

  CB = 1280
  tc_f = pl.pallas_call(
      _tc_f_body,
      grid=(NRB, pl.cdiv(N, CB)),
      in_specs=[pl.BlockSpec((RB, HID), lambda i, j: (i, 0)),
                pl.BlockSpec((CB, HID), lambda i, j: (j, 0))],
      out_specs=pl.BlockSpec((RB, CB), lambda i, j: (i, j)),
      out_shape=jax.ShapeDtypeStruct((N, N), f32),
  )
  s_ = tc_f(hs, hs)

  agg4 = agg_cs(hp4.reshape(2 * NPAD, W), src2, dst, zrows)

  tc_e = pl.pallas_call(
      _tc_e_body,
      grid=(NRB,),
      in_specs=[_halves((2, RB, W)), _halves((2, RB, W)), _rows((RB, 1)),
                _full((HID, IN_DIM)), _full((HID, IN_DIM)),
                _full((IN_DIM,)), _full((IN_DIM,))],
      out_specs=[_rows((RB, IN_DIM))] * 3,
      out_shape=[jax.ShapeDtypeStruct((N, IN_DIM), f32)] * 3,
  )
  x_s_hat, x_ns_hat, x_s_cf_hat = tc_e(agg4, hp4, dinv, Wa12, Wa22,
                                       ba12, ba22)

  return (z_s, z_ns, x_s_hat, x_ns_hat, x_s_cf_hat, s_)


# speedup vs baseline: 1.0998x; 1.0218x over previous
import functools

import jax
import jax.numpy as jnp
from jax import lax
from jax.experimental import pallas as pl
from jax.experimental.pallas import tpu as pltpu
from jax.experimental.pallas import tpu_sc as plsc

N = 10000
E = 320000
IN_DIM = 128
HID = 64

NC = 2
NS = 16
NPAD = 10240
RPT = NPAD // NS
EB = 112
EPAD = 322560
F0 = 215040
W = 128

RB = 512
NRB = (N + RB - 1) // RB

f32 = jnp.float32


def _make_sc_agg(col_split):
  mesh = plsc.VectorSubcoreMesh(core_axis_name="c", subcore_axis_name="s")
  if col_split:
    eps = EPAD // NS
    nblk = eps // EB
  else:
    eps0 = F0 // NS
    eps1 = (EPAD - F0) // NS
    nblk0 = eps0 // EB
    nblk1 = eps1 // EB

  @functools.partial(
      pl.kernel,
      out_type=jax.ShapeDtypeStruct((NC, NPAD, W), f32),
      mesh=mesh,
      scratch_types=[
          pltpu.VMEM((EB,), jnp.int32),
          pltpu.VMEM((EB,), jnp.int32),
          pltpu.VMEM((EB,), jnp.int32),
          pltpu.VMEM((EB,), jnp.int32),
          pltpu.VMEM((EB, W), f32),
          pltpu.VMEM((EB, W), f32),
          pltpu.VMEM_SHARED((NPAD, W), f32),
          pltpu.SemaphoreType.DMA,
          pltpu.SemaphoreType.DMA,
      ],
  )
  def agg(hp_hbm, src_hbm, dst_hbm, zeros_hbm, out_hbm,
          src0_v, src1_v, dst0_v, dst1_v, buf0_v, buf1_v, acc,
          gsem0, gsem1):
    c = lax.axis_index("c")
    s = lax.axis_index("s")
    pltpu.sync_copy(zeros_hbm, acc.at[pl.ds(s * RPT, RPT)])
    plsc.subcore_barrier()

    def offs(k):
      if col_split:
        base = s * eps + k * EB
        src_off = c * EPAD + base
      else:
        base = jnp.where(c == 0, s * eps0 + k * EB,
                         F0 + s * eps1 + k * EB)
        src_off = base
      return src_off, base

    def load_fire(k, src_v, dst_v, buf_v, sem):
      src_off, base = offs(k)
      pltpu.sync_copy(src_hbm.at[pl.ds(src_off, EB)], src_v)
      pltpu.sync_copy(dst_hbm.at[pl.ds(base, EB)], dst_v)
      pltpu.async_copy(hp_hbm.at[src_v], buf_v, sem)

    load_fire(0, src0_v, dst0_v, buf0_v, gsem0)

    def body(g, carry):
      load_fire(2 * g + 1, src1_v, dst1_v, buf1_v, gsem1)
      pltpu.make_async_copy(hp_hbm.at[pl.ds(0, EB)], buf0_v, gsem0).wait()
      pltpu.sync_copy(buf0_v, acc.at[dst0_v], add=True)
      load_fire(2 * g + 2, src0_v, dst0_v, buf0_v, gsem0)
      pltpu.make_async_copy(hp_hbm.at[pl.ds(0, EB)], buf1_v, gsem1).wait()
      pltpu.sync_copy(buf1_v, acc.at[dst1_v], add=True)
      return carry

    ngrp = (nblk // 2) if col_split else jnp.where(c == 0, nblk0 // 2,
                                                  nblk1 // 2)
    lax.fori_loop(0, ngrp, body, 0)
    pltpu.make_async_copy(hp_hbm.at[pl.ds(0, EB)], buf0_v, gsem0).wait()
    plsc.subcore_barrier()
    pltpu.sync_copy(acc.at[pl.ds(s * RPT, RPT)],
                    out_hbm.at[c, pl.ds(s * RPT, RPT)])

  return agg


def _make_sc_deg():
  mesh = plsc.VectorSubcoreMesh(core_axis_name="c", subcore_axis_name="s")
  eps = EPAD // (NC * NS)
  nblk = eps // EB

  @functools.partial(
      pl.kernel,
      out_type=jax.ShapeDtypeStruct((NC, NPAD, W), f32),
      mesh=mesh,
      scratch_types=[
          pltpu.VMEM((EB,), jnp.int32),
          pltpu.VMEM((EB, W), f32),
          pltpu.VMEM_SHARED((NPAD, W), f32),
      ],
  )
  def deg(ones_hbm, dst_hbm, zeros_hbm, out_hbm, dst_v, ones_v, acc):
    c = lax.axis_index("c")
    s = lax.axis_index("s")
    pltpu.sync_copy(zeros_hbm, acc.at[pl.ds(s * RPT, RPT)])
    pltpu.sync_copy(ones_hbm, ones_v)
    plsc.subcore_barrier()

    def body(k, carry):
      base = (c * NS + s) * eps + k * EB
      pltpu.sync_copy(dst_hbm.at[pl.ds(base, EB)], dst_v)
      pltpu.sync_copy(ones_v, acc.at[dst_v], add=True)
      return carry

    lax.fori_loop(0, nblk, body, 0)
    plsc.subcore_barrier()
    pltpu.sync_copy(acc.at[pl.ds(s * RPT, RPT)],
                    out_hbm.at[c, pl.ds(s * RPT, RPT)])

  return deg


def _full(shape):
  return pl.BlockSpec(shape, lambda i: tuple(0 for _ in shape))


def _rows(shape):
  rank = len(shape)
  return pl.BlockSpec(shape, lambda i: (i,) + tuple(0 for _ in range(rank - 1)))


def _halves(shape):
  return pl.BlockSpec(shape, lambda i: (0, i, 0))


def _cat2(ref):
  return jnp.concatenate([ref[0], ref[1]], axis=1)


def _sum2(ref):
  return ref[0] + ref[1]


def _tc_a_body(dp_ref, x_ref, we1_ref, dinv_ref, hp1_ref):
  deg = dp_ref[0, :, 0] + dp_ref[1, :, 0] + 1.0
  dinv = lax.rsqrt(deg)[:, None]
  x = x_ref[...]
  h = jnp.dot(x, we1_ref[...], preferred_element_type=f32)
  q0 = dinv * (1.0 - 2.0 * x[:, 0:1])
  wide = jnp.concatenate(
      [h * dinv, q0, jnp.zeros((RB, W - HID - 1), f32)], axis=1)
  dinv_ref[...] = dinv
  hp1_ref[...] = wide


def _tc_b_body(agg1_ref, hp1_ref, dinv_ref, we1_ref, w2_ref, be1_ref,
               hp2_ref):
  a = _sum2(agg1_ref) + hp1_ref[...]
  dinv = dinv_ref[...]
  pre1 = dinv * a[:, :64] + be1_ref[...][None, :]
  q = dinv * a[:, 64:65]
  h1 = jnp.maximum(pre1, 0.0)
  h1cf = jnp.maximum(pre1 + q * we1_ref[0:1, :], 0.0)
  w2 = w2_ref[...]
  h2 = jnp.concatenate(
      [jnp.dot(h1, w2, preferred_element_type=f32),
       jnp.dot(h1cf, w2[:, :32], preferred_element_type=f32),
       jnp.zeros((RB, 32), f32)], axis=1)
  hp2_ref[...] = dinv * h2


def _tc_c_body(agg2_ref, hp2_ref, dinv_ref, be2_ref, zs_ref, zns_ref,
               hp3_ref):
  a = _sum2(agg2_ref) + hp2_ref[...]
  dinv = dinv_ref[...]
  s2 = dinv * a
  be2 = be2_ref[...]
  z = s2[:, :64] + be2[None, :]
  z_s = z[:, :32]
  z_ns = z[:, 32:]
  z_s_cf = s2[:, 64:96] + be2[None, :32]
  zs_ref[...] = z_s
  zns_ref[...] = z_ns
  hp3_ref[...] = dinv * jnp.concatenate(
      [z_s, z_ns, z_s_cf, jnp.zeros((RB, 32), f32)], axis=1)


def _tc_d_body(agg3_ref, hp3_ref, dinv_ref, wa11_ref, wa21_ref, ws_ref,
               ba11_ref, ba21_ref, bs_ref, hs_ref, hp4_ref):
  v = dinv_ref[...] * (_sum2(agg3_ref) + hp3_ref[...])
  p_zs = v[:, 0:32]
  p_zns = v[:, 32:64]
  p_zscf = v[:, 64:96]
  wa11 = wa11_ref[...]
  ba11 = ba11_ref[...][None, :]
  u1 = jnp.maximum(jnp.dot(p_zs, wa11, preferred_element_type=f32) + ba11,
                   0.0)
  u2 = jnp.maximum(jnp.dot(p_zns, wa21_ref[...], preferred_element_type=f32)
                   + ba21_ref[...][None, :], 0.0)
  u3 = jnp.maximum(jnp.dot(p_zscf, wa11, preferred_element_type=f32) + ba11,
                   0.0)
  hs_ref[...] = (jnp.dot(p_zns, ws_ref[...], preferred_element_type=f32)
                 + bs_ref[...][None, :])
  dinv = dinv_ref[...]
  hp4_ref[0] = dinv * jnp.concatenate([u1, u2], axis=1)
  hp4_ref[1] = dinv * jnp.concatenate([u3, jnp.zeros((RB, 64), f32)], axis=1)


def _tc_e_body(agg4_ref, hp4_ref, dinv_ref, wa12_ref, wa22_ref,
               ba12_ref, ba22_ref, xs_ref, xns_ref, xscf_ref):
  va = dinv_ref[...] * (_cat2(agg4_ref) + _cat2(hp4_ref))
  wa12 = wa12_ref[...]
  ba12 = ba12_ref[...][None, :]
  xs_ref[...] = (jnp.dot(va[:, 0:64], wa12, preferred_element_type=f32)
                 + ba12)
  xns_ref[...] = (jnp.dot(va[:, 64:128], wa22_ref[...],
                          preferred_element_type=f32)
                  + ba22_ref[...][None, :])
  xscf_ref[...] = (jnp.dot(va[:, 128:192], wa12, preferred_element_type=f32)
                   + ba12)


def _tc_f_body(a_ref, b_ref, out_ref):
  out_ref[...] = lax.dot_general(
      a_ref[...], b_ref[...], (((1,), (1,)), ((), ())),
      preferred_element_type=f32)


def kernel(x, W_e1, b_e1, W_e2, b_e2, Wa11, ba11, Wa12, ba12, Wa21, ba21,
           Wa22, ba22, Ws, bs, edge_index):
  pad = jnp.full((EPAD - E,), N, jnp.int32)
  padd = N + 8 + (jnp.arange(EPAD - E, dtype=jnp.int32) % (NPAD - N - 8))
  ext = jnp.full((EB,), N, jnp.int32)
  src = jnp.concatenate([edge_index[0].astype(jnp.int32), pad, ext])
  dst = jnp.concatenate([edge_index[1].astype(jnp.int32), padd, ext + 8])
  src2 = jnp.concatenate([src[:EPAD], src[:EPAD] + NPAD, ext])
  zrows = jnp.zeros((RPT, W), f32)

  agg_es = _make_sc_agg(col_split=False)
  agg_cs = _make_sc_agg(col_split=True)

  degpart = _make_sc_deg()(jnp.ones((EB, W), f32), dst, zrows)

  tc_a = pl.pallas_call(
      _tc_a_body,
      grid=(NRB,),
      in_specs=[_halves((2, RB, W)),
                _rows((RB, IN_DIM)), _full((IN_DIM, HID))],
      out_specs=[_rows((RB, 1)), _rows((RB, W))],
      out_shape=[jax.ShapeDtypeStruct((N, 1), f32),
                 jax.ShapeDtypeStruct((NPAD, W), f32)],
  )
  dinv, hp1 = tc_a(degpart, x, W_e1)

  agg1 = agg_es(hp1, src, dst, zrows)

  tc_b = pl.pallas_call(
      _tc_b_body,
      grid=(NRB,),
      in_specs=[_halves((2, RB, W)), _rows((RB, W)), _rows((RB, 1)),
                _full((IN_DIM, HID)), _full((HID, HID)), _full((HID,))],
      out_specs=[_rows((RB, W))],
      out_shape=[jax.ShapeDtypeStruct((NPAD, W), f32)],
  )
  (hp2,) = tc_b(agg1, hp1, dinv, W_e1, W_e2, b_e1)

  agg2 = agg_es(hp2, src, dst, zrows)

  h = HID // 2
  tc_c = pl.pallas_call(
      _tc_c_body,
      grid=(NRB,),
      in_specs=[_halves((2, RB, W)), _rows((RB, W)), _rows((RB, 1)),
                _full((HID,))],
      out_specs=[_rows((RB, h)), _rows((RB, h)), _rows((RB, W))],
      out_shape=[jax.ShapeDtypeStruct((N, h), f32),
                 jax.ShapeDtypeStruct((N, h), f32),
                 jax.ShapeDtypeStruct((NPAD, W), f32)],
  )
  z_s, z_ns, hp3 = tc_c(agg2, hp2, dinv, b_e2)

  agg3 = agg_es(hp3, src, dst, zrows)

  tc_d = pl.pallas_call(
      _tc_d_body,
      grid=(NRB,),
      in_specs=[_halves((2, RB, W)), _rows((RB, W)), _rows((RB, 1)),
                _full((h, HID)), _full((h, HID)), _full((h, HID)),
                _full((HID,)), _full((HID,)), _full((HID,))],
      out_specs=[_rows((RB, HID)), _halves((2, RB, W))],
      out_shape=[jax.ShapeDtypeStruct((N, HID), f32),
                 jax.ShapeDtypeStruct((2, NPAD, W), f32)],
  )
  hs, hp4 = tc_d(agg3, hp3, dinv, Wa11, Wa21, Ws, ba11, ba21, bs)

  CB = 1280
  tc_f = pl.pallas_call(
      _tc_f_body,
      grid=(NRB, pl.cdiv(N, CB)),
      in_specs=[pl.BlockSpec((RB, HID), lambda i, j: (i, 0)),
                pl.BlockSpec((CB, HID), lambda i, j: (j, 0))],
      out_specs=pl.BlockSpec((RB, CB), lambda i, j: (i, j)),
      out_shape=jax.ShapeDtypeStruct((N, N), f32),
  )
  s_ = tc_f(hs, hs)

  agg4 = agg_cs(hp4.reshape(2 * NPAD, W), src2, dst, zrows)

  tc_e = pl.pallas_call(
      _tc_e_body,
      grid=(NRB,),
      in_specs=[_halves((2, RB, W)), _halves((2, RB, W)), _rows((RB, 1)),
                _full((HID, IN_DIM)), _full((HID, IN_DIM)),
                _full((IN_DIM,)), _full((IN_DIM,))],
      out_specs=[_rows((RB, IN_DIM))] * 3,
      out_shape=[jax.ShapeDtypeStruct((N, IN_DIM), f32)] * 3,
  )
  x_s_hat, x_ns_hat, x_s_cf_hat = tc_e(agg4, hp4, dinv, Wa12, Wa22,
                                       ba12, ba22)

  return (z_s, z_ns, x_s_hat, x_ns_hat, x_s_cf_hat, s_)
